# baseline (speedup 1.0000x reference)
import functools

import jax
import jax.numpy as jnp
from jax import lax
from jax.experimental import pallas as pl
from jax.experimental.pallas import tpu as pltpu
from jax.experimental.pallas import tpu_sc as plsc

D = 128
L = 16
NC = 2
NS = 16
NW = NC * NS
CHUNK = 32
K0 = 20
K1 = 0
NCHUNKS = NS * (K0 + K1)
BATCH_PAD = NCHUNKS * CHUNK
VPR = D // L


def _make_sc_call(fan_out, scale_val):
    mesh = plsc.VectorSubcoreMesh(core_axis_name="c", subcore_axis_name="s",
                                  num_cores=NC, num_subcores=NS)

    @functools.partial(
        pl.kernel,
        out_type=jax.ShapeDtypeStruct((BATCH_PAD, D), jnp.float32),
        mesh=mesh,
        scratch_types=[
            pltpu.VMEM((fan_out, CHUNK), jnp.int32),
            pltpu.VMEM((fan_out, CHUNK), jnp.int32),
            pltpu.VMEM((CHUNK, D), jnp.float32),
            pltpu.VMEM((CHUNK, D), jnp.float32),
            pltpu.SemaphoreType.DMA,
            pltpu.SemaphoreType.DMA,
            pltpu.SemaphoreType.DMA,
            pltpu.SemaphoreType.DMA,
        ],
    )
    def sc_call(feat_hbm, idx_hbm, out_hbm, idx0, idx1, acc0, acc1,
                sg0, sg1, so0, so1):
        c = lax.axis_index("c")
        s = lax.axis_index("s")
        kcnt = jnp.where(c == 0, K0, K1)
        base = jnp.where(c == 0, s * K0, NS * K0 + s * K1)
        idxs = (idx0, idx1)
        accs = (acc0, acc1)
        sgs = (sg0, sg1)
        sos = (so0, so1)
        zvec = jnp.zeros((L,), jnp.float32)

        def zero(acc):
            def zbody(r, carry):
                for v in range(VPR):
                    acc[r, pl.ds(v * L, L)] = zvec
                return carry
            lax.fori_loop(0, CHUNK, zbody, 0)

        def load_idx(k, idx_v):
            pltpu.sync_copy(idx_hbm.at[k], idx_v)

        def issue(idx_v, acc, sem):
            def ibody(n, carry):
                pltpu.async_copy(feat_hbm.at[idx_v.at[n]], acc, sem, add=True)
                return carry
            lax.fori_loop(0, fan_out, ibody, 0)

        def drain(idx_v, acc, sem):
            def dbody(n, carry):
                pltpu.make_async_copy(feat_hbm.at[idx_v.at[n]], acc,
                                      sem).wait()
                return carry
            lax.fori_loop(0, fan_out, dbody, 0)

        def scale_in_place(acc):
            def sbody(r, carry):
                for v in range(VPR):
                    acc[r, pl.ds(v * L, L)] = acc[r, pl.ds(v * L, L)] * scale_val
                return carry
            lax.fori_loop(0, CHUNK, sbody, 0)

        for slot in range(2):
            @pl.when(slot < kcnt)
            def _():
                load_idx(base + slot, idxs[slot])
                zero(accs[slot])
                issue(idxs[slot], accs[slot], sgs[slot])

        def step(t, carry):
            for slot in range(2):
                j = 2 * t + slot

                @pl.when(j < kcnt)
                def _():
                    k = base + j
                    drain(idxs[slot], accs[slot], sgs[slot])
                    scale_in_place(accs[slot])
                    cp = pltpu.async_copy(
                        accs[slot], out_hbm.at[pl.ds(k * CHUNK, CHUNK)],
                        sos[slot])

                    @pl.when(j + 2 < kcnt)
                    def _():
                        load_idx(k + 2, idxs[slot])

                    cp.wait()
                    zero(accs[slot])

                    @pl.when(j + 2 < kcnt)
                    def _():
                        issue(idxs[slot], accs[slot], sgs[slot])
            return carry

        lax.fori_loop(0, (jnp.maximum(kcnt, 1) + 1) // 2, step, 0)

    return sc_call


def kernel(features, neigh_idx, num_sample):
    n_nodes, d = features.shape
    batch, fan_out = neigh_idx.shape
    assert d == D
    idx = neigh_idx.astype(jnp.int32)
    pad = BATCH_PAD - batch
    if pad:
        idx = jnp.pad(idx, ((0, pad), (0, 0)))
    idx3 = idx.reshape(NCHUNKS, CHUNK, fan_out).transpose(0, 2, 1)
    scale = jnp.float32(1.0 / fan_out)
    sc_call = _make_sc_call(fan_out, scale)
    out = sc_call(features, idx3)
    return out[:batch]


# speedup vs baseline: 4.8172x; 4.8172x over previous
import functools

import jax
import jax.numpy as jnp
from jax import lax
from jax.experimental import pallas as pl
from jax.experimental.pallas import tpu as pltpu
from jax.experimental.pallas import tpu_sc as plsc

D = 128
L = 16
NC = 2
NS = 16
NW = NC * NS
CHUNK = 32
K0 = 10
K1 = 10
NCHUNKS = NS * (K0 + K1)
BATCH_PAD = NCHUNKS * CHUNK
VPR = D // L


def _make_sc_call(fan_out, scale_val):
    mesh = plsc.VectorSubcoreMesh(core_axis_name="c", subcore_axis_name="s",
                                  num_cores=NC, num_subcores=NS)

    @functools.partial(
        pl.kernel,
        out_type=jax.ShapeDtypeStruct((BATCH_PAD, D), jnp.float32),
        mesh=mesh,
        scratch_types=[
            pltpu.VMEM((fan_out, CHUNK), jnp.int32),
            pltpu.VMEM((fan_out, CHUNK), jnp.int32),
            pltpu.VMEM((CHUNK, D), jnp.float32),
            pltpu.VMEM((CHUNK, D), jnp.float32),
            pltpu.SemaphoreType.DMA,
            pltpu.SemaphoreType.DMA,
            pltpu.SemaphoreType.DMA,
            pltpu.SemaphoreType.DMA,
        ],
    )
    def sc_call(feat_hbm, idx_hbm, out_hbm, idx0, idx1, acc0, acc1,
                sg0, sg1, so0, so1):
        c = lax.axis_index("c")
        s = lax.axis_index("s")
        kcnt = jnp.where(c == 0, K0, K1)
        base = jnp.where(c == 0, s * K0, NS * K0 + s * K1)
        idxs = (idx0, idx1)
        accs = (acc0, acc1)
        sgs = (sg0, sg1)
        sos = (so0, so1)
        zvec = jnp.zeros((L,), jnp.float32)

        def zero(acc):
            def zbody(r, carry):
                for v in range(VPR):
                    acc[r, pl.ds(v * L, L)] = zvec
                return carry
            lax.fori_loop(0, CHUNK, zbody, 0)

        def load_idx(k, idx_v):
            pltpu.sync_copy(idx_hbm.at[k], idx_v)

        def issue(idx_v, acc, sem):
            def ibody(n, carry):
                pltpu.async_copy(feat_hbm.at[idx_v.at[n]], acc, sem, add=True)
                return carry
            lax.fori_loop(0, fan_out, ibody, 0)

        def drain(idx_v, acc, sem):
            def dbody(n, carry):
                pltpu.make_async_copy(feat_hbm.at[idx_v.at[n]], acc,
                                      sem).wait()
                return carry
            lax.fori_loop(0, fan_out, dbody, 0)

        def scale_in_place(acc):
            def sbody(r, carry):
                for v in range(VPR):
                    acc[r, pl.ds(v * L, L)] = acc[r, pl.ds(v * L, L)] * scale_val
                return carry
            lax.fori_loop(0, CHUNK, sbody, 0)

        for slot in range(2):
            @pl.when(slot < kcnt)
            def _():
                load_idx(base + slot, idxs[slot])
                zero(accs[slot])
                issue(idxs[slot], accs[slot], sgs[slot])

        def step(t, carry):
            for slot in range(2):
                j = 2 * t + slot

                @pl.when(j < kcnt)
                def _():
                    k = base + j
                    drain(idxs[slot], accs[slot], sgs[slot])
                    scale_in_place(accs[slot])
                    cp = pltpu.async_copy(
                        accs[slot], out_hbm.at[pl.ds(k * CHUNK, CHUNK)],
                        sos[slot])

                    @pl.when(j + 2 < kcnt)
                    def _():
                        load_idx(k + 2, idxs[slot])

                    cp.wait()
                    zero(accs[slot])

                    @pl.when(j + 2 < kcnt)
                    def _():
                        issue(idxs[slot], accs[slot], sgs[slot])
            return carry

        lax.fori_loop(0, (jnp.maximum(kcnt, 1) + 1) // 2, step, 0)

    return sc_call


def kernel(features, neigh_idx, num_sample):
    n_nodes, d = features.shape
    batch, fan_out = neigh_idx.shape
    assert d == D
    idx = neigh_idx.astype(jnp.int32)
    pad = BATCH_PAD - batch
    if pad:
        fill = (jnp.arange(pad * fan_out, dtype=jnp.int32) % n_nodes
                ).reshape(pad, fan_out)
        idx = jnp.concatenate([idx, fill], axis=0)
    idx3 = idx.reshape(NCHUNKS, CHUNK, fan_out).transpose(0, 2, 1)
    scale = jnp.float32(1.0 / fan_out)
    sc_call = _make_sc_call(fan_out, scale)
    out = sc_call(features, idx3)
    return out[:batch]


# speedup vs baseline: 5.0930x; 1.0573x over previous
import functools

import jax
import jax.numpy as jnp
from jax import lax
from jax.experimental import pallas as pl
from jax.experimental.pallas import tpu as pltpu
from jax.experimental.pallas import tpu_sc as plsc

D = 128
L = 16
NC = 2
NS = 16
NW = NC * NS
CHUNK = 64
K = 5
NCHUNKS = NW * K
BATCH_PAD = NCHUNKS * CHUNK
VPR = D // L


def _make_sc_call(fan_out, scale_val):
    mesh = plsc.VectorSubcoreMesh(core_axis_name="c", subcore_axis_name="s",
                                  num_cores=NC, num_subcores=NS)

    @functools.partial(
        pl.kernel,
        out_type=jax.ShapeDtypeStruct((BATCH_PAD, D), jnp.float32),
        mesh=mesh,
        scratch_types=[
            pltpu.VMEM((K, fan_out, CHUNK), jnp.int32),
            pltpu.VMEM((CHUNK, D), jnp.float32),
            pltpu.VMEM((CHUNK, D), jnp.float32),
            pltpu.VMEM((CHUNK, D), jnp.float32),
            pltpu.VMEM((CHUNK, D), jnp.float32),
            pltpu.SemaphoreType.DMA,
            pltpu.SemaphoreType.DMA,
            pltpu.SemaphoreType.DMA,
            pltpu.SemaphoreType.DMA,
        ],
    )
    def sc_call(feat_hbm, idx_hbm, out_hbm, idx_v, acc0, acc1, st0, st1,
                sg0, sg1, so0, so1):
        c = lax.axis_index("c")
        s = lax.axis_index("s")
        wid = s * NC + c
        base = wid * K
        accs = (acc0, acc1)
        stages = (st0, st1)
        sgs = (sg0, sg1)
        sos = (so0, so1)
        zvec = jnp.zeros((L,), jnp.float32)

        pltpu.sync_copy(idx_hbm.at[pl.ds(base, K)], idx_v)

        def zero(acc):
            def zbody(r, carry):
                for v in range(VPR):
                    acc[r, pl.ds(v * L, L)] = zvec
                return carry
            lax.fori_loop(0, CHUNK, zbody, 0)

        def issue(j, acc, sem):
            def ibody(n, carry):
                pltpu.async_copy(feat_hbm.at[idx_v.at[j, n]], acc, sem,
                                 add=True)
                return carry
            lax.fori_loop(0, fan_out, ibody, 0)

        def drain(j, acc, sem):
            def dbody(n, carry):
                pltpu.make_async_copy(feat_hbm.at[idx_v.at[j, n]], acc,
                                      sem).wait()
                return carry
            lax.fori_loop(0, fan_out, dbody, 0)

        def scale_to(acc, stage):
            def sbody(r, carry):
                for v in range(VPR):
                    stage[r, pl.ds(v * L, L)] = (
                        acc[r, pl.ds(v * L, L)] * scale_val)
                return carry
            lax.fori_loop(0, CHUNK, sbody, 0)

        def out_copy(j, stage, sem):
            return pltpu.make_async_copy(
                stage, out_hbm.at[pl.ds((base + j) * CHUNK, CHUNK)], sem)

        for slot in range(2):
            zero(accs[slot])
            issue(slot, accs[slot], sgs[slot])

        for j in range(K):
            p = j % 2
            drain(j, accs[p], sgs[p])
            if j >= 2:
                out_copy(j - 2, stages[p], sos[p]).wait()
            scale_to(accs[p], stages[p])
            out_copy(j, stages[p], sos[p]).start()
            if j + 2 < K:
                zero(accs[p])
                issue(j + 2, accs[p], sgs[p])

        for j in (K - 2, K - 1):
            out_copy(j, stages[j % 2], sos[j % 2]).wait()

    return sc_call


def kernel(features, neigh_idx, num_sample):
    n_nodes, d = features.shape
    batch, fan_out = neigh_idx.shape
    assert d == D
    idx = neigh_idx.astype(jnp.int32)
    pad = BATCH_PAD - batch
    if pad:
        fill = (jnp.arange(pad * fan_out, dtype=jnp.int32) % n_nodes
                ).reshape(pad, fan_out)
        idx = jnp.concatenate([idx, fill], axis=0)
    idx3 = idx.reshape(NCHUNKS, CHUNK, fan_out).transpose(0, 2, 1)
    scale = jnp.float32(1.0 / fan_out)
    sc_call = _make_sc_call(fan_out, scale)
    out = sc_call(features, idx3)
    return out[:batch]
